# Initial kernel scaffold; baseline (speedup 1.0000x reference)
import jax
import jax.numpy as jnp
from jax.experimental import pallas as pl


def kernel(x, edge_index, W1, a1_src, a1_dst, W2, a2_src, a2_dst):
    raise NotImplementedError("write your pallas kernel here")


# speedup vs baseline: 11.0916x; 11.0916x over previous
import functools

import jax
import jax.numpy as jnp
from jax import lax
from jax.experimental import pallas as pl
from jax.experimental.pallas import tpu as pltpu
from jax.experimental.pallas import tpu_sc as plsc

N = 10000
NPAD = 10240
E = 320000
D = 128
CORES = 2
SUB = 16
EPT = E // (CORES * SUB)
CH = 16
EB = 2000
STRIPE = NPAD // SUB


def _tc_proj(x, W, avec):
    rows = x.shape[0]
    blk = 1024 if rows % 1024 == 0 else 1000
    grid = rows // blk

    def body(x_ref, w_ref, a_ref, z_ref, zsd_ref):
        z = jnp.dot(x_ref[...], w_ref[...], preferred_element_type=jnp.float32)
        z_ref[...] = z
        zsd_ref[...] = jnp.dot(z, a_ref[...], preferred_element_type=jnp.float32)

    return pl.pallas_call(
        body,
        grid=(grid,),
        in_specs=[
            pl.BlockSpec((blk, D), lambda i: (i, 0)),
            pl.BlockSpec((D, D), lambda i: (0, 0)),
            pl.BlockSpec((D, 2), lambda i: (0, 0)),
        ],
        out_specs=[
            pl.BlockSpec((blk, D), lambda i: (i, 0)),
            pl.BlockSpec((blk, 2), lambda i: (i, 0)),
        ],
        out_shape=[
            jax.ShapeDtypeStruct((rows, D), jnp.float32),
            jax.ShapeDtypeStruct((rows, 2), jnp.float32),
        ],
    )(x, W, avec)


def _tc_combine(part, den, maxes, W, avec, apply_act):
    blk = 1024
    grid = NPAD // blk
    project = W is not None

    def body(part_ref, den_ref, m_ref, *rest):
        if project:
            w_ref, a_ref, z_ref, zsd_ref = rest
        else:
            (h_ref,) = rest
        m0 = m_ref[0]
        m1 = m_ref[SUB]
        mm = jnp.maximum(m0, m1)
        s0 = jnp.exp(m0 - mm)
        s1 = jnp.exp(m1 - mm)
        num = s0 * part_ref[0] + s1 * part_ref[1]
        den = s0 * den_ref[0] + s1 * den_ref[1]
        h = num / (den + 1e-9)
        if apply_act:
            h = jnp.where(h > 0, h, jnp.exp(jnp.minimum(h, 0.0)) - 1.0)
        if project:
            z = jnp.dot(h, w_ref[...], preferred_element_type=jnp.float32)
            z_ref[...] = z
            zsd_ref[...] = jnp.dot(z, a_ref[...], preferred_element_type=jnp.float32)
        else:
            h_ref[...] = h

    in_specs = [
        pl.BlockSpec((2, blk, D), lambda i: (0, i, 0)),
        pl.BlockSpec((2, blk, 1), lambda i: (0, i, 0)),
        pl.BlockSpec(memory_space=pltpu.MemorySpace.SMEM),
    ]
    if project:
        in_specs += [
            pl.BlockSpec((D, D), lambda i: (0, 0)),
            pl.BlockSpec((D, 2), lambda i: (0, 0)),
        ]
        out_specs = [
            pl.BlockSpec((blk, D), lambda i: (i, 0)),
            pl.BlockSpec((blk, 2), lambda i: (i, 0)),
        ]
        out_shape = [
            jax.ShapeDtypeStruct((NPAD, D), jnp.float32),
            jax.ShapeDtypeStruct((NPAD, 2), jnp.float32),
        ]
        args = (part, den, maxes, W, avec)
    else:
        out_specs = [pl.BlockSpec((blk, D), lambda i: (i, 0))]
        out_shape = [jax.ShapeDtypeStruct((NPAD, D), jnp.float32)]
        args = (part, den, maxes)

    return pl.pallas_call(
        body,
        grid=(grid,),
        in_specs=in_specs,
        out_specs=out_specs,
        out_shape=out_shape,
    )(*args)


@functools.cache
def _make_sc_edge(table_rows):
    mesh = plsc.VectorSubcoreMesh(core_axis_name="c", subcore_axis_name="s")

    @functools.partial(
        pl.kernel,
        out_type=(
            jax.ShapeDtypeStruct((CORES, NPAD, D), jnp.float32),
            jax.ShapeDtypeStruct((CORES * NPAD,), jnp.float32),
            jax.ShapeDtypeStruct((CORES * SUB,), jnp.float32),
        ),
        mesh=mesh,
        scratch_types=[
            pltpu.VMEM((NPAD * 2,), jnp.float32),
            pltpu.VMEM((EB,), jnp.int32),
            pltpu.VMEM((EB,), jnp.int32),
            pltpu.VMEM((16,), jnp.float32),
            pltpu.VMEM((CH, D), jnp.float32),
            pltpu.VMEM((16,), jnp.float32),
            pltpu.VMEM((16, 16), jnp.float32),
            pltpu.VMEM((32, D), jnp.float32),
            pltpu.VMEM((STRIPE,), jnp.float32),
            pltpu.VMEM_SHARED((NPAD, D), jnp.float32),
            pltpu.VMEM_SHARED((NPAD,), jnp.float32),
            pltpu.VMEM_SHARED((16, 16), jnp.float32),
            pltpu.SemaphoreType.DMA,
        ],
        compiler_params=pltpu.CompilerParams(needs_layout_passes=False),
    )
    def sc_edge(z_hbm, zsd_hbm, edge_hbm, part_hbm, den_hbm, max_hbm,
                zsd_v, src_v, dst_v, exb_v, rows_v,
                mbuf_v, allm_v, zbuf_v, dbuf_v,
                shared_out, shared_den, shared_max, sem):
        cid = lax.axis_index("c")
        sid = lax.axis_index("s")
        base = (cid * SUB + sid) * EPT

        pltpu.sync_copy(zsd_hbm, zsd_v.at[pl.ds(0, table_rows * 2)])

        zero16 = jnp.zeros((16,), jnp.float32)

        def zrow(i, c):
            for j in range(D // 16):
                zbuf_v[i, pl.ds(j * 16, 16)] = zero16
            return c

        lax.fori_loop(0, 32, zrow, 0)

        def zden(i, c):
            dbuf_v[pl.ds(i * 16, 16)] = zero16
            return c

        lax.fori_loop(0, STRIPE // 16, zden, 0)

        for b in range(STRIPE // 32):
            pltpu.sync_copy(zbuf_v, shared_out.at[pl.ds(sid * STRIPE + b * 32, 32)])
        pltpu.sync_copy(dbuf_v, shared_den.at[pl.ds(sid * STRIPE, STRIPE)])

        oi16 = jnp.ones((16,), jnp.int32)

        def edge_logits(i):
            s16 = src_v[pl.ds(i * 16, 16)]
            d16 = dst_v[pl.ds(i * 16, 16)]
            ea = plsc.load_gather(zsd_v, [s16 * 2])
            eb = plsc.load_gather(zsd_v, [d16 * 2 + oi16])
            e = ea + eb
            return s16, d16, jnp.where(e >= 0.0, e, e * jnp.float32(0.2))

        def p1blk(b, m):
            pltpu.sync_copy(edge_hbm.at[pl.ds(base + b * EB, EB)], src_v)
            pltpu.sync_copy(edge_hbm.at[pl.ds(E + base + b * EB, EB)], dst_v)

            def p1(i, m2):
                _, _, e = edge_logits(i)
                return jnp.maximum(m2, e)

            return lax.fori_loop(0, EB // 16, p1, m)

        m = lax.fori_loop(0, EPT // EB, p1blk,
                          jnp.full((16,), -3.4e38, jnp.float32))
        mbuf_v[...] = m
        pltpu.sync_copy(mbuf_v, shared_max.at[sid])
        plsc.subcore_barrier()
        pltpu.sync_copy(shared_max, allm_v)
        mm = allm_v[0, :]
        for r in range(1, 16):
            mm = jnp.maximum(mm, allm_v[r, :])
        gmax = jnp.max(mm)

        @pl.when(sid == 0)
        def _():
            mbuf_v[...] = zero16 + gmax
            pltpu.sync_copy(mbuf_v, max_hbm.at[pl.ds(cid * SUB, 16)])

        def p23blk(b, c):
            pltpu.sync_copy(edge_hbm.at[pl.ds(base + b * EB, EB)], src_v)
            pltpu.sync_copy(edge_hbm.at[pl.ds(E + base + b * EB, EB)], dst_v)

            def p23(i, c2):
                s16, d16, e = edge_logits(i)
                ex = jnp.exp(e - gmax)
                exb_v[...] = ex
                pltpu.async_copy(z_hbm.at[s16], rows_v, sem).wait()
                for r in range(CH):
                    exr = ex[r]
                    for j in range(D // 16):
                        rows_v[r, pl.ds(j * 16, 16)] = rows_v[r, pl.ds(j * 16, 16)] * exr
                pltpu.sync_copy(exb_v, shared_den.at[d16], add=True)
                pltpu.sync_copy(rows_v, shared_out.at[d16], add=True)
                return c2

            return lax.fori_loop(0, EB // 16, p23, c)

        lax.fori_loop(0, EPT // EB, p23blk, 0)
        plsc.subcore_barrier()

        for b in range(STRIPE // 32):
            r0 = sid * STRIPE + b * 32
            pltpu.sync_copy(shared_out.at[pl.ds(r0, 32)], zbuf_v)
            pltpu.sync_copy(zbuf_v, part_hbm.at[cid, pl.ds(r0, 32)])
        pltpu.sync_copy(shared_den.at[pl.ds(sid * STRIPE, STRIPE)], dbuf_v)
        pltpu.sync_copy(dbuf_v, den_hbm.at[pl.ds(cid * NPAD + sid * STRIPE, STRIPE)])

    return sc_edge


def kernel(x, edge_index, W1, a1_src, a1_dst, W2, a2_src, a2_dst):
    avec1 = jnp.stack([a1_src, a1_dst], axis=1)
    avec2 = jnp.stack([a2_src, a2_dst], axis=1)
    edge_flat = jnp.reshape(edge_index, (2 * E,))

    z1, zsd1 = _tc_proj(x, W1, avec1)
    part1, den1, max1 = _make_sc_edge(N)(z1, jnp.reshape(zsd1, (-1,)), edge_flat)
    z2, zsd2 = _tc_combine(part1, jnp.reshape(den1, (CORES, NPAD, 1)), max1,
                           W2, avec2, apply_act=True)
    part2, den2, max2 = _make_sc_edge(NPAD)(z2, jnp.reshape(zsd2, (-1,)), edge_flat)
    (h,) = _tc_combine(part2, jnp.reshape(den2, (CORES, NPAD, 1)), max2,
                       None, None, apply_act=False)
    return h[:N]


# speedup vs baseline: 19.9545x; 1.7991x over previous
import functools

import jax
import jax.numpy as jnp
from jax import lax
from jax.experimental import pallas as pl
from jax.experimental.pallas import tpu as pltpu
from jax.experimental.pallas import tpu_sc as plsc

N = 10000
NPAD = 10240
E = 320000
D = 128
CORES = 2
SUB = 16
EPT = E // (CORES * SUB)
CH = 16
EB = 2000
STRIPE = NPAD // SUB


def _tc_proj(x, W, avec):
    rows = x.shape[0]
    blk = 1024 if rows % 1024 == 0 else 1000
    grid = rows // blk

    def body(x_ref, w_ref, a_ref, z_ref, zsd_ref):
        z = jnp.dot(x_ref[...], w_ref[...], preferred_element_type=jnp.float32)
        z_ref[...] = z
        zsd_ref[...] = jnp.dot(z, a_ref[...], preferred_element_type=jnp.float32)

    return pl.pallas_call(
        body,
        grid=(grid,),
        in_specs=[
            pl.BlockSpec((blk, D), lambda i: (i, 0)),
            pl.BlockSpec((D, D), lambda i: (0, 0)),
            pl.BlockSpec((D, 2), lambda i: (0, 0)),
        ],
        out_specs=[
            pl.BlockSpec((blk, D), lambda i: (i, 0)),
            pl.BlockSpec((blk, 2), lambda i: (i, 0)),
        ],
        out_shape=[
            jax.ShapeDtypeStruct((rows, D), jnp.float32),
            jax.ShapeDtypeStruct((rows, 2), jnp.float32),
        ],
    )(x, W, avec)


def _tc_combine(part, den, maxes, W, avec, apply_act):
    blk = 1000
    grid = N // blk
    project = W is not None

    def body(part_ref, den_ref, m_ref, *rest):
        if project:
            w_ref, a_ref, z_ref, zsd_ref = rest
        else:
            (h_ref,) = rest
        m0 = m_ref[0]
        m1 = m_ref[SUB]
        mm = jnp.maximum(m0, m1)
        s0 = jnp.exp(m0 - mm)
        s1 = jnp.exp(m1 - mm)
        num = s0 * part_ref[0] + s1 * part_ref[1]
        den = s0 * den_ref[0] + s1 * den_ref[1]
        h = num / (den + 1e-9)
        if apply_act:
            h = jnp.where(h > 0, h, jnp.exp(jnp.minimum(h, 0.0)) - 1.0)
        if project:
            z = jnp.dot(h, w_ref[...], preferred_element_type=jnp.float32)
            z_ref[...] = z
            zsd_ref[...] = jnp.dot(z, a_ref[...], preferred_element_type=jnp.float32)
        else:
            h_ref[...] = h

    in_specs = [
        pl.BlockSpec((2, blk, D), lambda i: (0, i, 0)),
        pl.BlockSpec((2, blk, 1), lambda i: (0, i, 0)),
        pl.BlockSpec(memory_space=pltpu.MemorySpace.SMEM),
    ]
    if project:
        in_specs += [
            pl.BlockSpec((D, D), lambda i: (0, 0)),
            pl.BlockSpec((D, 2), lambda i: (0, 0)),
        ]
        out_specs = [
            pl.BlockSpec((blk, D), lambda i: (i, 0)),
            pl.BlockSpec((blk, 2), lambda i: (i, 0)),
        ]
        out_shape = [
            jax.ShapeDtypeStruct((N, D), jnp.float32),
            jax.ShapeDtypeStruct((N, 2), jnp.float32),
        ]
        args = (part, den, maxes, W, avec)
    else:
        out_specs = [pl.BlockSpec((blk, D), lambda i: (i, 0))]
        out_shape = [jax.ShapeDtypeStruct((N, D), jnp.float32)]
        args = (part, den, maxes)

    return pl.pallas_call(
        body,
        grid=(grid,),
        in_specs=in_specs,
        out_specs=out_specs,
        out_shape=out_shape,
    )(*args)


@functools.cache
def _make_sc_edge():
    mesh = plsc.VectorSubcoreMesh(core_axis_name="c", subcore_axis_name="s")
    NCH = EPT // CH

    @functools.partial(
        pl.kernel,
        out_type=(
            jax.ShapeDtypeStruct((CORES, NPAD, D), jnp.float32),
            jax.ShapeDtypeStruct((CORES * NPAD,), jnp.float32),
            jax.ShapeDtypeStruct((CORES * SUB,), jnp.float32),
        ),
        mesh=mesh,
        scratch_types=[
            pltpu.VMEM((N * 2,), jnp.float32),
            pltpu.VMEM((EPT,), jnp.int32),
            pltpu.VMEM((EPT,), jnp.int32),
            pltpu.VMEM((16,), jnp.float32),
            pltpu.VMEM((2 * CH, D), jnp.float32),
            pltpu.VMEM((16,), jnp.float32),
            pltpu.VMEM((16, 16), jnp.float32),
            pltpu.VMEM((STRIPE,), jnp.float32),
            pltpu.VMEM_SHARED((NPAD, D), jnp.float32),
            pltpu.VMEM_SHARED((NPAD,), jnp.float32),
            pltpu.VMEM_SHARED((16, 16), jnp.float32),
            pltpu.SemaphoreType.DMA,
            pltpu.SemaphoreType.DMA,
        ],
        compiler_params=pltpu.CompilerParams(needs_layout_passes=False),
    )
    def sc_edge(z_hbm, zsd_hbm, edge_hbm, part_hbm, den_hbm, max_hbm,
                zsd_v, src_v, dst_v, exb_v, rows_v,
                mbuf_v, allm_v, dbuf_v,
                shared_out, shared_den, shared_max, sem0, sem1):
        cid = lax.axis_index("c")
        sid = lax.axis_index("s")
        base = (cid * SUB + sid) * EPT
        sems = (sem0, sem1)

        pltpu.sync_copy(zsd_hbm, zsd_v)
        pltpu.sync_copy(edge_hbm.at[pl.ds(base, EPT)], src_v)
        pltpu.sync_copy(edge_hbm.at[pl.ds(E + base, EPT)], dst_v)

        zero16 = jnp.zeros((16,), jnp.float32)

        def zrow(i, c):
            for j in range(D // 16):
                rows_v[i, pl.ds(j * 16, 16)] = zero16
            return c

        lax.fori_loop(0, 2 * CH, zrow, 0)

        def zden(i, c):
            dbuf_v[pl.ds(i * 16, 16)] = zero16
            return c

        lax.fori_loop(0, STRIPE // 16, zden, 0)

        for b in range(STRIPE // (2 * CH)):
            pltpu.sync_copy(
                rows_v, shared_out.at[pl.ds(sid * STRIPE + b * 2 * CH, 2 * CH)])
        pltpu.sync_copy(dbuf_v, shared_den.at[pl.ds(sid * STRIPE, STRIPE)])

        oi16 = jnp.ones((16,), jnp.int32)

        def edge_logits(i):
            s16 = src_v[pl.ds(i * 16, 16)]
            d16 = dst_v[pl.ds(i * 16, 16)]
            ea = plsc.load_gather(zsd_v, [s16 * 2])
            eb = plsc.load_gather(zsd_v, [d16 * 2 + oi16])
            e = ea + eb
            return s16, d16, jnp.where(e >= 0.0, e, e * jnp.float32(0.2))

        def p1(i, m):
            _, _, e = edge_logits(i)
            return jnp.maximum(m, e)

        m = lax.fori_loop(0, NCH, p1, jnp.full((16,), -3.4e38, jnp.float32))
        mbuf_v[...] = m
        pltpu.sync_copy(mbuf_v, shared_max.at[sid])
        plsc.subcore_barrier()
        pltpu.sync_copy(shared_max, allm_v)
        mm = allm_v[0, :]
        for r in range(1, 16):
            mm = jnp.maximum(mm, allm_v[r, :])
        gmax = jnp.max(mm)

        @pl.when(sid == 0)
        def _():
            mbuf_v[...] = zero16 + gmax
            pltpu.sync_copy(mbuf_v, max_hbm.at[pl.ds(cid * SUB, 16)])

        def issue_gather(i, buf):
            s16 = src_v[pl.ds(i * 16, 16)]
            pltpu.async_copy(
                z_hbm.at[s16], rows_v.at[pl.ds(buf * CH, CH)], sems[buf])

        def process(i, buf):
            nxt = i + 1

            @pl.when(nxt < NCH)
            def _():
                issue_gather(nxt, 1 - buf)

            pltpu.make_async_copy(
                z_hbm.at[pl.ds(0, CH)],
                rows_v.at[pl.ds(buf * CH, CH)], sems[buf]).wait()
            _, d16, e = edge_logits(i)
            ex = jnp.exp(e - gmax)
            exb_v[...] = ex
            for r in range(CH):
                exr = ex[r]
                rr = buf * CH + r
                for j in range(D // 16):
                    rows_v[rr, pl.ds(j * 16, 16)] = (
                        rows_v[rr, pl.ds(j * 16, 16)] * exr)
            pltpu.sync_copy(exb_v, shared_den.at[d16], add=True)
            pltpu.sync_copy(rows_v.at[pl.ds(buf * CH, CH)],
                            shared_out.at[d16], add=True)

        issue_gather(0, 0)

        def p23(i, c):
            @pl.when(lax.rem(i, 2) == 0)
            def _():
                process(i, 0)

            @pl.when(lax.rem(i, 2) == 1)
            def _():
                process(i, 1)

            return c

        lax.fori_loop(0, NCH, p23, 0)
        plsc.subcore_barrier()

        for b in range(STRIPE // (2 * CH)):
            r0 = sid * STRIPE + b * 2 * CH
            pltpu.sync_copy(shared_out.at[pl.ds(r0, 2 * CH)], rows_v)
            pltpu.sync_copy(rows_v, part_hbm.at[cid, pl.ds(r0, 2 * CH)])
        pltpu.sync_copy(shared_den.at[pl.ds(sid * STRIPE, STRIPE)], dbuf_v)
        pltpu.sync_copy(dbuf_v, den_hbm.at[pl.ds(cid * NPAD + sid * STRIPE, STRIPE)])

    return sc_edge


def kernel(x, edge_index, W1, a1_src, a1_dst, W2, a2_src, a2_dst):
    avec1 = jnp.stack([a1_src, a1_dst], axis=1)
    avec2 = jnp.stack([a2_src, a2_dst], axis=1)
    edge_flat = jnp.reshape(edge_index, (2 * E,))

    z1, zsd1 = _tc_proj(x, W1, avec1)
    part1, den1, max1 = _make_sc_edge()(z1, jnp.reshape(zsd1, (-1,)), edge_flat)
    z2, zsd2 = _tc_combine(part1, jnp.reshape(den1, (CORES, NPAD, 1)), max1,
                           W2, avec2, apply_act=True)
    part2, den2, max2 = _make_sc_edge()(z2, jnp.reshape(zsd2, (-1,)), edge_flat)
    (h,) = _tc_combine(part2, jnp.reshape(den2, (CORES, NPAD, 1)), max2,
                       None, None, apply_act=False)
    return h


# speedup vs baseline: 20.9063x; 1.0477x over previous
import functools

import jax
import jax.numpy as jnp
from jax import lax
from jax.experimental import pallas as pl
from jax.experimental.pallas import tpu as pltpu
from jax.experimental.pallas import tpu_sc as plsc

N = 10000
NPAD = 10240
E = 320000
D = 128
CORES = 2
SUB = 16
EPT = E // (CORES * SUB)
CH = 16
EB = 2000
STRIPE = NPAD // SUB


def _tc_proj(x, W, avec):
    rows = x.shape[0]
    blk = 1024 if rows % 1024 == 0 else 1000
    grid = rows // blk

    def body(x_ref, w_ref, a_ref, z_ref, zsd_ref):
        z = jnp.dot(x_ref[...], w_ref[...], preferred_element_type=jnp.float32)
        z_ref[...] = z
        zsd_ref[...] = jnp.dot(z, a_ref[...], preferred_element_type=jnp.float32)

    return pl.pallas_call(
        body,
        grid=(grid,),
        in_specs=[
            pl.BlockSpec((blk, D), lambda i: (i, 0)),
            pl.BlockSpec((D, D), lambda i: (0, 0)),
            pl.BlockSpec((D, 2), lambda i: (0, 0)),
        ],
        out_specs=[
            pl.BlockSpec((blk, D), lambda i: (i, 0)),
            pl.BlockSpec((blk, 2), lambda i: (i, 0)),
        ],
        out_shape=[
            jax.ShapeDtypeStruct((rows, D), jnp.float32),
            jax.ShapeDtypeStruct((rows, 2), jnp.float32),
        ],
    )(x, W, avec)


def _tc_combine(part, den, maxes, W, avec, apply_act):
    blk = 1000
    grid = N // blk
    project = W is not None

    def body(part_ref, den_ref, m_ref, *rest):
        if project:
            w_ref, a_ref, z_ref, zsd_ref = rest
        else:
            (h_ref,) = rest
        m0 = m_ref[0]
        m1 = m_ref[SUB]
        mm = jnp.maximum(m0, m1)
        s0 = jnp.exp(m0 - mm)
        s1 = jnp.exp(m1 - mm)
        num = s0 * part_ref[0] + s1 * part_ref[1]
        den = s0 * den_ref[0] + s1 * den_ref[1]
        h = num / (den + 1e-9)
        if apply_act:
            h = jnp.where(h > 0, h, jnp.exp(jnp.minimum(h, 0.0)) - 1.0)
        if project:
            z = jnp.dot(h, w_ref[...], preferred_element_type=jnp.float32)
            z_ref[...] = z
            zsd_ref[...] = jnp.dot(z, a_ref[...], preferred_element_type=jnp.float32)
        else:
            h_ref[...] = h

    in_specs = [
        pl.BlockSpec((2, blk, D), lambda i: (0, i, 0)),
        pl.BlockSpec((2, blk, 1), lambda i: (0, i, 0)),
        pl.BlockSpec(memory_space=pltpu.MemorySpace.SMEM),
    ]
    if project:
        in_specs += [
            pl.BlockSpec((D, D), lambda i: (0, 0)),
            pl.BlockSpec((D, 2), lambda i: (0, 0)),
        ]
        out_specs = [
            pl.BlockSpec((blk, D), lambda i: (i, 0)),
            pl.BlockSpec((blk, 2), lambda i: (i, 0)),
        ]
        out_shape = [
            jax.ShapeDtypeStruct((N, D), jnp.float32),
            jax.ShapeDtypeStruct((N, 2), jnp.float32),
        ]
        args = (part, den, maxes, W, avec)
    else:
        out_specs = [pl.BlockSpec((blk, D), lambda i: (i, 0))]
        out_shape = [jax.ShapeDtypeStruct((N, D), jnp.float32)]
        args = (part, den, maxes)

    return pl.pallas_call(
        body,
        grid=(grid,),
        in_specs=in_specs,
        out_specs=out_specs,
        out_shape=out_shape,
    )(*args)


@functools.cache
def _make_sc_edge():
    mesh = plsc.VectorSubcoreMesh(core_axis_name="c", subcore_axis_name="s")
    NCH = EPT // CH

    @functools.partial(
        pl.kernel,
        out_type=(
            jax.ShapeDtypeStruct((CORES, NPAD, D), jnp.float32),
            jax.ShapeDtypeStruct((CORES * NPAD,), jnp.float32),
            jax.ShapeDtypeStruct((CORES * SUB,), jnp.float32),
        ),
        mesh=mesh,
        scratch_types=[
            pltpu.VMEM((N * 2,), jnp.float32),
            pltpu.VMEM((EPT,), jnp.int32),
            pltpu.VMEM((EPT,), jnp.int32),
            pltpu.VMEM((2, 16), jnp.float32),
            pltpu.VMEM((2 * CH, D), jnp.float32),
            pltpu.VMEM((16,), jnp.float32),
            pltpu.VMEM((16, 16), jnp.float32),
            pltpu.VMEM((STRIPE,), jnp.float32),
            pltpu.VMEM_SHARED((NPAD, D), jnp.float32),
            pltpu.VMEM_SHARED((NPAD,), jnp.float32),
            pltpu.VMEM_SHARED((16, 16), jnp.float32),
            pltpu.SemaphoreType.DMA,
            pltpu.SemaphoreType.DMA,
            pltpu.SemaphoreType.DMA,
            pltpu.SemaphoreType.DMA,
            pltpu.SemaphoreType.DMA,
            pltpu.SemaphoreType.DMA,
        ],
        compiler_params=pltpu.CompilerParams(needs_layout_passes=False),
    )
    def sc_edge(z_hbm, zsd_hbm, edge_hbm, part_hbm, den_hbm, max_hbm,
                zsd_v, src_v, dst_v, exb_v, rows_v,
                mbuf_v, allm_v, dbuf_v,
                shared_out, shared_den, shared_max,
                sem0, sem1, ssem0, ssem1, dsem0, dsem1):
        cid = lax.axis_index("c")
        sid = lax.axis_index("s")
        base = (cid * SUB + sid) * EPT
        sems = (sem0, sem1)
        ssems = (ssem0, ssem1)
        dsems = (dsem0, dsem1)

        pltpu.sync_copy(zsd_hbm, zsd_v)
        pltpu.sync_copy(edge_hbm.at[pl.ds(base, EPT)], src_v)
        pltpu.sync_copy(edge_hbm.at[pl.ds(E + base, EPT)], dst_v)

        zero16 = jnp.zeros((16,), jnp.float32)

        def zrow(i, c):
            for j in range(D // 16):
                rows_v[i, pl.ds(j * 16, 16)] = zero16
            return c

        lax.fori_loop(0, 2 * CH, zrow, 0)

        def zden(i, c):
            dbuf_v[pl.ds(i * 16, 16)] = zero16
            return c

        lax.fori_loop(0, STRIPE // 16, zden, 0)

        for b in range(STRIPE // (2 * CH)):
            pltpu.sync_copy(
                rows_v, shared_out.at[pl.ds(sid * STRIPE + b * 2 * CH, 2 * CH)])
        pltpu.sync_copy(dbuf_v, shared_den.at[pl.ds(sid * STRIPE, STRIPE)])

        oi16 = jnp.ones((16,), jnp.int32)

        def edge_logits(i):
            s16 = src_v[pl.ds(i * 16, 16)]
            d16 = dst_v[pl.ds(i * 16, 16)]
            ea = plsc.load_gather(zsd_v, [s16 * 2])
            eb = plsc.load_gather(zsd_v, [d16 * 2 + oi16])
            e = ea + eb
            return s16, d16, jnp.where(e >= 0.0, e, e * jnp.float32(0.2))

        def p1(i, m):
            _, _, e = edge_logits(i)
            return jnp.maximum(m, e)

        m = lax.fori_loop(0, NCH, p1, jnp.full((16,), -3.4e38, jnp.float32))
        mbuf_v[...] = m
        pltpu.sync_copy(mbuf_v, shared_max.at[sid])
        plsc.subcore_barrier()
        pltpu.sync_copy(shared_max, allm_v)
        mm = allm_v[0, :]
        for r in range(1, 16):
            mm = jnp.maximum(mm, allm_v[r, :])
        gmax = jnp.max(mm)

        @pl.when(sid == 0)
        def _():
            mbuf_v[...] = zero16 + gmax
            pltpu.sync_copy(mbuf_v, max_hbm.at[pl.ds(cid * SUB, 16)])

        def issue_gather(i, buf):
            s16 = src_v[pl.ds(i * 16, 16)]
            pltpu.async_copy(
                z_hbm.at[s16], rows_v.at[pl.ds(buf * CH, CH)], sems[buf])

        def wait_scatters(buf):
            pltpu.make_async_copy(
                rows_v.at[pl.ds(buf * CH, CH)],
                shared_out.at[pl.ds(0, CH)], ssems[buf]).wait()
            pltpu.make_async_copy(
                exb_v.at[buf], shared_den.at[pl.ds(0, 16)],
                dsems[buf]).wait()

        def process(i, buf):
            nxt = i + 1

            @pl.when(i > 0)
            def _():
                wait_scatters(1 - buf)

            @pl.when(nxt < NCH)
            def _():
                issue_gather(nxt, 1 - buf)

            pltpu.make_async_copy(
                z_hbm.at[pl.ds(0, CH)],
                rows_v.at[pl.ds(buf * CH, CH)], sems[buf]).wait()
            _, d16, e = edge_logits(i)
            ex = jnp.exp(e - gmax)
            exb_v[buf] = ex
            for r in range(CH):
                exr = ex[r]
                rr = buf * CH + r
                for j in range(D // 16):
                    rows_v[rr, pl.ds(j * 16, 16)] = (
                        rows_v[rr, pl.ds(j * 16, 16)] * exr)
            pltpu.async_copy(exb_v.at[buf], shared_den.at[d16],
                             dsems[buf], add=True)
            pltpu.async_copy(rows_v.at[pl.ds(buf * CH, CH)],
                             shared_out.at[d16], ssems[buf], add=True)

        issue_gather(0, 0)

        def p23(i, c):
            @pl.when(lax.rem(i, 2) == 0)
            def _():
                process(i, 0)

            @pl.when(lax.rem(i, 2) == 1)
            def _():
                process(i, 1)

            return c

        lax.fori_loop(0, NCH, p23, 0)
        wait_scatters((NCH - 1) % 2)
        plsc.subcore_barrier()

        for b in range(STRIPE // (2 * CH)):
            r0 = sid * STRIPE + b * 2 * CH
            pltpu.sync_copy(shared_out.at[pl.ds(r0, 2 * CH)], rows_v)
            pltpu.sync_copy(rows_v, part_hbm.at[cid, pl.ds(r0, 2 * CH)])
        pltpu.sync_copy(shared_den.at[pl.ds(sid * STRIPE, STRIPE)], dbuf_v)
        pltpu.sync_copy(dbuf_v, den_hbm.at[pl.ds(cid * NPAD + sid * STRIPE, STRIPE)])

    return sc_edge


def kernel(x, edge_index, W1, a1_src, a1_dst, W2, a2_src, a2_dst):
    avec1 = jnp.stack([a1_src, a1_dst], axis=1)
    avec2 = jnp.stack([a2_src, a2_dst], axis=1)
    edge_flat = jnp.reshape(edge_index, (2 * E,))

    z1, zsd1 = _tc_proj(x, W1, avec1)
    part1, den1, max1 = _make_sc_edge()(z1, jnp.reshape(zsd1, (-1,)), edge_flat)
    z2, zsd2 = _tc_combine(part1, jnp.reshape(den1, (CORES, NPAD, 1)), max1,
                           W2, avec2, apply_act=True)
    part2, den2, max2 = _make_sc_edge()(z2, jnp.reshape(zsd2, (-1,)), edge_flat)
    (h,) = _tc_combine(part2, jnp.reshape(den2, (CORES, NPAD, 1)), max2,
                       None, None, apply_act=False)
    return h


# speedup vs baseline: 28.6712x; 1.3714x over previous
import functools

import jax
import jax.numpy as jnp
from jax import lax
from jax.experimental import pallas as pl
from jax.experimental.pallas import tpu as pltpu
from jax.experimental.pallas import tpu_sc as plsc

N = 10000
NPAD = 10240
E = 320000
D = 128
CORES = 2
SUB = 16
EPT = E // (CORES * SUB)
CH = 80
EB = 2000
NB = EPT // EB
CPB = EB // CH
STRIPE = NPAD // SUB


def _tc_proj(x, W, avec):
    rows = x.shape[0]
    blk = 1024 if rows % 1024 == 0 else 1000
    grid = rows // blk

    def body(x_ref, w_ref, a_ref, z_ref, zsd_ref):
        z = jnp.dot(x_ref[...], w_ref[...], preferred_element_type=jnp.float32)
        z_ref[...] = z
        zsd_ref[...] = jnp.dot(z, a_ref[...], preferred_element_type=jnp.float32)

    return pl.pallas_call(
        body,
        grid=(grid,),
        in_specs=[
            pl.BlockSpec((blk, D), lambda i: (i, 0)),
            pl.BlockSpec((D, D), lambda i: (0, 0)),
            pl.BlockSpec((D, 2), lambda i: (0, 0)),
        ],
        out_specs=[
            pl.BlockSpec((blk, D), lambda i: (i, 0)),
            pl.BlockSpec((blk, 2), lambda i: (i, 0)),
        ],
        out_shape=[
            jax.ShapeDtypeStruct((rows, D), jnp.float32),
            jax.ShapeDtypeStruct((rows, 2), jnp.float32),
        ],
    )(x, W, avec)


def _tc_combine(part, den, maxes, W, avec, apply_act):
    blk = 1000
    grid = N // blk
    project = W is not None

    def body(part_ref, den_ref, m_ref, *rest):
        if project:
            w_ref, a_ref, z_ref, zsd_ref = rest
        else:
            (h_ref,) = rest
        m0 = m_ref[0]
        m1 = m_ref[SUB]
        mm = jnp.maximum(m0, m1)
        s0 = jnp.exp(m0 - mm)
        s1 = jnp.exp(m1 - mm)
        num = s0 * part_ref[0] + s1 * part_ref[1]
        den = s0 * den_ref[0] + s1 * den_ref[1]
        h = num / (den + 1e-9)
        if apply_act:
            h = jnp.where(h > 0, h, jnp.exp(jnp.minimum(h, 0.0)) - 1.0)
        if project:
            z = jnp.dot(h, w_ref[...], preferred_element_type=jnp.float32)
            z_ref[...] = z
            zsd_ref[...] = jnp.dot(z, a_ref[...], preferred_element_type=jnp.float32)
        else:
            h_ref[...] = h

    in_specs = [
        pl.BlockSpec((2, blk, D), lambda i: (0, i, 0)),
        pl.BlockSpec((2, blk, 1), lambda i: (0, i, 0)),
        pl.BlockSpec(memory_space=pltpu.MemorySpace.SMEM),
    ]
    if project:
        in_specs += [
            pl.BlockSpec((D, D), lambda i: (0, 0)),
            pl.BlockSpec((D, 2), lambda i: (0, 0)),
        ]
        out_specs = [
            pl.BlockSpec((blk, D), lambda i: (i, 0)),
            pl.BlockSpec((blk, 2), lambda i: (i, 0)),
        ]
        out_shape = [
            jax.ShapeDtypeStruct((N, D), jnp.float32),
            jax.ShapeDtypeStruct((N, 2), jnp.float32),
        ]
        args = (part, den, maxes, W, avec)
    else:
        out_specs = [pl.BlockSpec((blk, D), lambda i: (i, 0))]
        out_shape = [jax.ShapeDtypeStruct((N, D), jnp.float32)]
        args = (part, den, maxes)

    return pl.pallas_call(
        body,
        grid=(grid,),
        in_specs=in_specs,
        out_specs=out_specs,
        out_shape=out_shape,
    )(*args)


@functools.cache
def _make_sc_edge():
    mesh = plsc.VectorSubcoreMesh(core_axis_name="c", subcore_axis_name="s")

    @functools.partial(
        pl.kernel,
        out_type=(
            jax.ShapeDtypeStruct((CORES, NPAD, D), jnp.float32),
            jax.ShapeDtypeStruct((CORES * NPAD,), jnp.float32),
            jax.ShapeDtypeStruct((CORES * SUB,), jnp.float32),
        ),
        mesh=mesh,
        scratch_types=[
            pltpu.VMEM((N * 2,), jnp.float32),
            pltpu.VMEM((EB,), jnp.int32),
            pltpu.VMEM((EB,), jnp.int32),
            pltpu.VMEM((CH,), jnp.int32),
            pltpu.VMEM((CH,), jnp.int32),
            pltpu.VMEM((CH,), jnp.float32),
            pltpu.VMEM((CH,), jnp.float32),
            pltpu.VMEM((2 * CH, D), jnp.float32),
            pltpu.VMEM((16,), jnp.float32),
            pltpu.VMEM((16, 16), jnp.float32),
            pltpu.VMEM((STRIPE,), jnp.float32),
            pltpu.VMEM_SHARED((NPAD, D), jnp.float32),
            pltpu.VMEM_SHARED((NPAD,), jnp.float32),
            pltpu.VMEM_SHARED((16, 16), jnp.float32),
            pltpu.SemaphoreType.DMA,
            pltpu.SemaphoreType.DMA,
            pltpu.SemaphoreType.DMA,
            pltpu.SemaphoreType.DMA,
            pltpu.SemaphoreType.DMA,
            pltpu.SemaphoreType.DMA,
        ],
        compiler_params=pltpu.CompilerParams(needs_layout_passes=False),
    )
    def sc_edge(z_hbm, zsd_hbm, edge_hbm, part_hbm, den_hbm, max_hbm,
                zsd_v, src_v, dst_v, idx0_v, idx1_v, exb0_v, exb1_v, rows_v,
                mbuf_v, allm_v, dbuf_v,
                shared_out, shared_den, shared_max,
                sem0, sem1, ssem0, ssem1, dsem0, dsem1):
        cid = lax.axis_index("c")
        sid = lax.axis_index("s")
        base = (cid * SUB + sid) * EPT
        sems = (sem0, sem1)
        ssems = (ssem0, ssem1)
        dsems = (dsem0, dsem1)
        idxs = (idx0_v, idx1_v)
        exbs = (exb0_v, exb1_v)

        pltpu.sync_copy(zsd_hbm, zsd_v)

        zero16 = jnp.zeros((16,), jnp.float32)

        def zrow(i, c):
            for j in range(D // 16):
                rows_v[i, pl.ds(j * 16, 16)] = zero16
            return c

        lax.fori_loop(0, 2 * CH, zrow, 0)

        def zden(i, c):
            dbuf_v[pl.ds(i * 16, 16)] = zero16
            return c

        lax.fori_loop(0, STRIPE // 16, zden, 0)

        for b in range(STRIPE // (2 * CH)):
            pltpu.sync_copy(
                rows_v, shared_out.at[pl.ds(sid * STRIPE + b * 2 * CH, 2 * CH)])
        pltpu.sync_copy(dbuf_v, shared_den.at[pl.ds(sid * STRIPE, STRIPE)])

        oi16 = jnp.ones((16,), jnp.int32)

        def stage_block(b):
            pltpu.sync_copy(edge_hbm.at[pl.ds(base + b * EB, EB)], src_v)
            pltpu.sync_copy(edge_hbm.at[pl.ds(E + base + b * EB, EB)], dst_v)

        def edge_logits(j):
            s16 = src_v[pl.ds(j * 16, 16)]
            d16 = dst_v[pl.ds(j * 16, 16)]
            ea = plsc.load_gather(zsd_v, [s16 * 2])
            eb = plsc.load_gather(zsd_v, [d16 * 2 + oi16])
            e = ea + eb
            return s16, d16, jnp.where(e >= 0.0, e, e * jnp.float32(0.2))

        def p1blk(b, m):
            stage_block(b)

            def p1(j, m2):
                _, _, e = edge_logits(j)
                return jnp.maximum(m2, e)

            return lax.fori_loop(0, EB // 16, p1, m)

        m = lax.fori_loop(0, NB, p1blk, jnp.full((16,), -3.4e38, jnp.float32))
        mbuf_v[...] = m
        pltpu.sync_copy(mbuf_v, shared_max.at[sid])
        plsc.subcore_barrier()
        pltpu.sync_copy(shared_max, allm_v)
        mm = allm_v[0, :]
        for r in range(1, 16):
            mm = jnp.maximum(mm, allm_v[r, :])
        gmax = jnp.max(mm)

        @pl.when(sid == 0)
        def _():
            mbuf_v[...] = zero16 + gmax
            pltpu.sync_copy(mbuf_v, max_hbm.at[pl.ds(cid * SUB, 16)])

        def issue_gather(i, buf):
            pltpu.async_copy(z_hbm.at[src_v.at[pl.ds(i * CH, CH)]],
                             rows_v.at[pl.ds(buf * CH, CH)], sems[buf])

        def wait_scatters(buf):
            pltpu.make_async_copy(
                rows_v.at[pl.ds(buf * CH, CH)],
                shared_out.at[pl.ds(0, CH)], ssems[buf]).wait()
            pltpu.make_async_copy(
                exbs[buf], shared_den.at[pl.ds(0, CH)], dsems[buf]).wait()

        def process(i, buf):
            nxt = i + 1

            @pl.when(i > 0)
            def _():
                wait_scatters(1 - buf)

            @pl.when(nxt < CPB)
            def _():
                issue_gather(nxt, 1 - buf)

            exs = []
            for j in range(CH // 16):
                _, d16, e = edge_logits(i * (CH // 16) + j)
                ex = jnp.exp(e - gmax)
                exs.append(ex)
                idxs[buf][pl.ds(j * 16, 16)] = d16
                exbs[buf][pl.ds(j * 16, 16)] = ex
            pltpu.make_async_copy(
                z_hbm.at[pl.ds(0, CH)],
                rows_v.at[pl.ds(buf * CH, CH)], sems[buf]).wait()
            for r in range(CH):
                exr = exs[r // 16][r % 16]
                rr = buf * CH + r
                for j in range(D // 16):
                    rows_v[rr, pl.ds(j * 16, 16)] = (
                        rows_v[rr, pl.ds(j * 16, 16)] * exr)
            pltpu.async_copy(exbs[buf], shared_den.at[idxs[buf]],
                             dsems[buf], add=True)
            pltpu.async_copy(rows_v.at[pl.ds(buf * CH, CH)],
                             shared_out.at[idxs[buf]], ssems[buf], add=True)

        def p23blk(b, c):
            stage_block(b)
            issue_gather(0, 0)

            def p23(i, c2):
                @pl.when(lax.rem(i, 2) == 0)
                def _():
                    process(i, 0)

                @pl.when(lax.rem(i, 2) == 1)
                def _():
                    process(i, 1)

                return c2

            lax.fori_loop(0, CPB, p23, 0)
            wait_scatters((CPB - 1) % 2)
            return c

        lax.fori_loop(0, NB, p23blk, 0)
        plsc.subcore_barrier()

        for b in range(STRIPE // (2 * CH)):
            r0 = sid * STRIPE + b * 2 * CH
            pltpu.sync_copy(shared_out.at[pl.ds(r0, 2 * CH)], rows_v)
            pltpu.sync_copy(rows_v, part_hbm.at[cid, pl.ds(r0, 2 * CH)])
        pltpu.sync_copy(shared_den.at[pl.ds(sid * STRIPE, STRIPE)], dbuf_v)
        pltpu.sync_copy(dbuf_v, den_hbm.at[pl.ds(cid * NPAD + sid * STRIPE, STRIPE)])

    return sc_edge


def kernel(x, edge_index, W1, a1_src, a1_dst, W2, a2_src, a2_dst):
    avec1 = jnp.stack([a1_src, a1_dst], axis=1)
    avec2 = jnp.stack([a2_src, a2_dst], axis=1)
    edge_flat = jnp.reshape(edge_index, (2 * E,))

    z1, zsd1 = _tc_proj(x, W1, avec1)
    part1, den1, max1 = _make_sc_edge()(z1, jnp.reshape(zsd1, (-1,)), edge_flat)
    z2, zsd2 = _tc_combine(part1, jnp.reshape(den1, (CORES, NPAD, 1)), max1,
                           W2, avec2, apply_act=True)
    part2, den2, max2 = _make_sc_edge()(z2, jnp.reshape(zsd2, (-1,)), edge_flat)
    (h,) = _tc_combine(part2, jnp.reshape(den2, (CORES, NPAD, 1)), max2,
                       None, None, apply_act=False)
    return h


# speedup vs baseline: 29.3623x; 1.0241x over previous
import functools

import jax
import jax.numpy as jnp
from jax import lax
from jax.experimental import pallas as pl
from jax.experimental.pallas import tpu as pltpu
from jax.experimental.pallas import tpu_sc as plsc

N = 10000
NPAD = 10240
E = 320000
D = 128
CORES = 2
SUB = 16
EPT = E // (CORES * SUB)
CH = 80
EB = 2000
NB = EPT // EB
CPB = EB // CH
STRIPE = NPAD // SUB


def _tc_proj(x, W, avec):
    rows = x.shape[0]
    blk = 1024 if rows % 1024 == 0 else 1000
    grid = rows // blk

    def body(x_ref, w_ref, a_ref, z_ref, zsd_ref):
        z = jnp.dot(x_ref[...], w_ref[...], preferred_element_type=jnp.float32)
        z_ref[...] = z
        zsd_ref[...] = jnp.dot(z, a_ref[...], preferred_element_type=jnp.float32)

    return pl.pallas_call(
        body,
        grid=(grid,),
        in_specs=[
            pl.BlockSpec((blk, D), lambda i: (i, 0)),
            pl.BlockSpec((D, D), lambda i: (0, 0)),
            pl.BlockSpec((D, 2), lambda i: (0, 0)),
        ],
        out_specs=[
            pl.BlockSpec((blk, D), lambda i: (i, 0)),
            pl.BlockSpec((blk, 2), lambda i: (i, 0)),
        ],
        out_shape=[
            jax.ShapeDtypeStruct((rows, D), jnp.float32),
            jax.ShapeDtypeStruct((rows, 2), jnp.float32),
        ],
    )(x, W, avec)


def _tc_combine(part, den, maxes, W, avec, apply_act):
    blk = 1000
    grid = N // blk
    project = W is not None

    def body(part_ref, den_ref, m_ref, *rest):
        if project:
            w_ref, a_ref, z_ref, zsd_ref = rest
        else:
            (h_ref,) = rest
        m0 = m_ref[0]
        m1 = m_ref[SUB]
        mm = jnp.maximum(m0, m1)
        s0 = jnp.exp(m0 - mm)
        s1 = jnp.exp(m1 - mm)
        num = s0 * part_ref[0] + s1 * part_ref[1]
        den = s0 * den_ref[0] + s1 * den_ref[1]
        h = num / (den + 1e-9)
        if apply_act:
            h = jnp.where(h > 0, h, jnp.exp(jnp.minimum(h, 0.0)) - 1.0)
        if project:
            z = jnp.dot(h, w_ref[...], preferred_element_type=jnp.float32)
            z_ref[...] = z
            zsd_ref[...] = jnp.dot(z, a_ref[...], preferred_element_type=jnp.float32)
        else:
            h_ref[...] = h

    in_specs = [
        pl.BlockSpec((2, blk, D), lambda i: (0, i, 0)),
        pl.BlockSpec((2, blk, 1), lambda i: (0, i, 0)),
        pl.BlockSpec(memory_space=pltpu.MemorySpace.SMEM),
    ]
    if project:
        in_specs += [
            pl.BlockSpec((D, D), lambda i: (0, 0)),
            pl.BlockSpec((D, 2), lambda i: (0, 0)),
        ]
        out_specs = [
            pl.BlockSpec((blk, D), lambda i: (i, 0)),
            pl.BlockSpec((blk, 2), lambda i: (i, 0)),
        ]
        out_shape = [
            jax.ShapeDtypeStruct((N, D), jnp.float32),
            jax.ShapeDtypeStruct((N, 2), jnp.float32),
        ]
        args = (part, den, maxes, W, avec)
    else:
        out_specs = [pl.BlockSpec((blk, D), lambda i: (i, 0))]
        out_shape = [jax.ShapeDtypeStruct((N, D), jnp.float32)]
        args = (part, den, maxes)

    return pl.pallas_call(
        body,
        grid=(grid,),
        in_specs=in_specs,
        out_specs=out_specs,
        out_shape=out_shape,
    )(*args)


@functools.cache
def _make_sc_edge():
    mesh = plsc.VectorSubcoreMesh(core_axis_name="c", subcore_axis_name="s")

    @functools.partial(
        pl.kernel,
        out_type=(
            jax.ShapeDtypeStruct((CORES, NPAD, D), jnp.float32),
            jax.ShapeDtypeStruct((CORES * NPAD,), jnp.float32),
            jax.ShapeDtypeStruct((CORES * SUB,), jnp.float32),
        ),
        mesh=mesh,
        scratch_types=[
            pltpu.VMEM((N * 2,), jnp.float32),
            pltpu.VMEM((EB,), jnp.int32),
            pltpu.VMEM((EB,), jnp.int32),
            pltpu.VMEM((CH,), jnp.int32),
            pltpu.VMEM((CH,), jnp.int32),
            pltpu.VMEM((CH,), jnp.float32),
            pltpu.VMEM((CH,), jnp.float32),
            pltpu.VMEM((2 * CH, D), jnp.float32),
            pltpu.VMEM((16,), jnp.float32),
            pltpu.VMEM((STRIPE,), jnp.float32),
            pltpu.VMEM_SHARED((NPAD, D), jnp.float32),
            pltpu.VMEM_SHARED((NPAD,), jnp.float32),
            pltpu.SemaphoreType.DMA,
            pltpu.SemaphoreType.DMA,
            pltpu.SemaphoreType.DMA,
            pltpu.SemaphoreType.DMA,
            pltpu.SemaphoreType.DMA,
            pltpu.SemaphoreType.DMA,
        ],
        compiler_params=pltpu.CompilerParams(needs_layout_passes=False),
    )
    def sc_edge(z_hbm, zsd_hbm, edge_hbm, part_hbm, den_hbm, max_hbm,
                zsd_v, src_v, dst_v, idx0_v, idx1_v, exb0_v, exb1_v, rows_v,
                mbuf_v, dbuf_v,
                shared_out, shared_den,
                sem0, sem1, ssem0, ssem1, dsem0, dsem1):
        cid = lax.axis_index("c")
        sid = lax.axis_index("s")
        base = (cid * SUB + sid) * EPT
        sems = (sem0, sem1)
        ssems = (ssem0, ssem1)
        dsems = (dsem0, dsem1)
        idxs = (idx0_v, idx1_v)
        exbs = (exb0_v, exb1_v)

        pltpu.sync_copy(zsd_hbm, zsd_v)

        zero16 = jnp.zeros((16,), jnp.float32)

        def zrow(i, c):
            for j in range(D // 16):
                rows_v[i, pl.ds(j * 16, 16)] = zero16
            return c

        lax.fori_loop(0, 2 * CH, zrow, 0)

        def zden(i, c):
            dbuf_v[pl.ds(i * 16, 16)] = zero16
            return c

        lax.fori_loop(0, STRIPE // 16, zden, 0)

        for b in range(STRIPE // (2 * CH)):
            pltpu.sync_copy(
                rows_v, shared_out.at[pl.ds(sid * STRIPE + b * 2 * CH, 2 * CH)])
        pltpu.sync_copy(dbuf_v, shared_den.at[pl.ds(sid * STRIPE, STRIPE)])

        oi16 = jnp.ones((16,), jnp.int32)

        def stage_block(b):
            pltpu.sync_copy(edge_hbm.at[pl.ds(base + b * EB, EB)], src_v)
            pltpu.sync_copy(edge_hbm.at[pl.ds(E + base + b * EB, EB)], dst_v)

        def edge_logits(j):
            s16 = src_v[pl.ds(j * 16, 16)]
            d16 = dst_v[pl.ds(j * 16, 16)]
            ea = plsc.load_gather(zsd_v, [s16 * 2])
            eb = plsc.load_gather(zsd_v, [d16 * 2 + oi16])
            e = ea + eb
            return s16, d16, jnp.where(e >= 0.0, e, e * jnp.float32(0.2))

        def pmax(i, mv):
            return jnp.maximum(mv, zsd_v[pl.ds(i * 16, 16)])

        mx = lax.fori_loop(0, (2 * N) // 16, pmax,
                           jnp.full((16,), -3.4e38, jnp.float32))
        two = jnp.max(mx) * jnp.float32(2.0)
        gmax = jnp.where(two >= 0.0, two, two * jnp.float32(0.2))
        plsc.subcore_barrier()

        @pl.when(sid == 0)
        def _():
            mbuf_v[...] = zero16 + gmax
            pltpu.sync_copy(mbuf_v, max_hbm.at[pl.ds(cid * SUB, 16)])

        def issue_gather(i, buf):
            pltpu.async_copy(z_hbm.at[src_v.at[pl.ds(i * CH, CH)]],
                             rows_v.at[pl.ds(buf * CH, CH)], sems[buf])

        def wait_scatters(buf):
            pltpu.make_async_copy(
                rows_v.at[pl.ds(buf * CH, CH)],
                shared_out.at[pl.ds(0, CH)], ssems[buf]).wait()
            pltpu.make_async_copy(
                exbs[buf], shared_den.at[pl.ds(0, CH)], dsems[buf]).wait()

        def process(i, buf):
            nxt = i + 1

            @pl.when(i > 0)
            def _():
                wait_scatters(1 - buf)

            @pl.when(nxt < CPB)
            def _():
                issue_gather(nxt, 1 - buf)

            exs = []
            for j in range(CH // 16):
                _, d16, e = edge_logits(i * (CH // 16) + j)
                ex = jnp.exp(e - gmax)
                exs.append(ex)
                idxs[buf][pl.ds(j * 16, 16)] = d16
                exbs[buf][pl.ds(j * 16, 16)] = ex
            pltpu.make_async_copy(
                z_hbm.at[pl.ds(0, CH)],
                rows_v.at[pl.ds(buf * CH, CH)], sems[buf]).wait()
            for r in range(CH):
                exr = exs[r // 16][r % 16]
                rr = buf * CH + r
                for j in range(D // 16):
                    rows_v[rr, pl.ds(j * 16, 16)] = (
                        rows_v[rr, pl.ds(j * 16, 16)] * exr)
            pltpu.async_copy(exbs[buf], shared_den.at[idxs[buf]],
                             dsems[buf], add=True)
            pltpu.async_copy(rows_v.at[pl.ds(buf * CH, CH)],
                             shared_out.at[idxs[buf]], ssems[buf], add=True)

        def p23blk(b, c):
            stage_block(b)
            issue_gather(0, 0)

            def p23(i, c2):
                @pl.when(lax.rem(i, 2) == 0)
                def _():
                    process(i, 0)

                @pl.when(lax.rem(i, 2) == 1)
                def _():
                    process(i, 1)

                return c2

            lax.fori_loop(0, CPB, p23, 0)
            wait_scatters((CPB - 1) % 2)
            return c

        lax.fori_loop(0, NB, p23blk, 0)
        plsc.subcore_barrier()

        for b in range(STRIPE // (2 * CH)):
            r0 = sid * STRIPE + b * 2 * CH
            pltpu.sync_copy(shared_out.at[pl.ds(r0, 2 * CH)], rows_v)
            pltpu.sync_copy(rows_v, part_hbm.at[cid, pl.ds(r0, 2 * CH)])
        pltpu.sync_copy(shared_den.at[pl.ds(sid * STRIPE, STRIPE)], dbuf_v)
        pltpu.sync_copy(dbuf_v, den_hbm.at[pl.ds(cid * NPAD + sid * STRIPE, STRIPE)])

    return sc_edge


def kernel(x, edge_index, W1, a1_src, a1_dst, W2, a2_src, a2_dst):
    avec1 = jnp.stack([a1_src, a1_dst], axis=1)
    avec2 = jnp.stack([a2_src, a2_dst], axis=1)
    edge_flat = jnp.reshape(edge_index, (2 * E,))

    z1, zsd1 = _tc_proj(x, W1, avec1)
    part1, den1, max1 = _make_sc_edge()(z1, jnp.reshape(zsd1, (-1,)), edge_flat)
    z2, zsd2 = _tc_combine(part1, jnp.reshape(den1, (CORES, NPAD, 1)), max1,
                           W2, avec2, apply_act=True)
    part2, den2, max2 = _make_sc_edge()(z2, jnp.reshape(zsd2, (-1,)), edge_flat)
    (h,) = _tc_combine(part2, jnp.reshape(den2, (CORES, NPAD, 1)), max2,
                       None, None, apply_act=False)
    return h
